# Initial kernel scaffold; baseline (speedup 1.0000x reference)
import jax
import jax.numpy as jnp
from jax.experimental import pallas as pl


def kernel(x, y, beta, bias):
    raise NotImplementedError("write your pallas kernel here")


# speedup vs baseline: 180.6309x; 180.6309x over previous
import functools

import jax
import jax.numpy as jnp
from jax import lax
from jax.experimental import pallas as pl
from jax.experimental.pallas import tpu as pltpu
from jax.experimental.pallas import tpu_sc as plsc

_LANES = 16
_NSUB = 16
_CHUNK = 4096


def _sc_gather_body(n_per_sub, beta_hbm, bias_hbm, y_hbm, out_hbm,
                    table_v, idx_v, vals_v):
    cid = lax.axis_index("c")
    sid = lax.axis_index("s")

    @pl.when(cid == 0)
    def _():
        pltpu.sync_copy(beta_hbm, table_v)

    @pl.when(cid != 0)
    def _():
        pltpu.sync_copy(bias_hbm, table_v)

    base = sid * n_per_sub

    def chunk_body(g, _):
        off = base + g * _CHUNK
        pltpu.sync_copy(y_hbm.at[pl.ds(off, _CHUNK)], idx_v)

        def inner(i, _):
            iv = idx_v[pl.ds(i * _LANES, _LANES)]
            vals_v[pl.ds(i * _LANES, _LANES)] = plsc.load_gather(
                table_v, [iv])
            return 0

        lax.fori_loop(0, _CHUNK // _LANES, inner, 0, unroll=4)
        pltpu.sync_copy(vals_v, out_hbm.at[cid, pl.ds(off, _CHUNK)])
        return 0

    lax.fori_loop(0, n_per_sub // _CHUNK, chunk_body, 0)


@functools.partial(jax.jit, static_argnames=("n", "n_tab"))
def _sc_gather(beta_f, bias_f, y_f, *, n, n_tab):
    n_per_sub = n // _NSUB
    body = functools.partial(_sc_gather_body, n_per_sub)
    return pl.kernel(
        body,
        out_type=jax.ShapeDtypeStruct((2, n), jnp.float32),
        mesh=plsc.VectorSubcoreMesh(core_axis_name="c", subcore_axis_name="s"),
        scratch_types=[
            pltpu.VMEM((n_tab,), jnp.float32),
            pltpu.VMEM((_CHUNK,), jnp.int32),
            pltpu.VMEM((_CHUNK,), jnp.float32),
        ],
        compiler_params=pltpu.CompilerParams(needs_layout_passes=False),
        name="sc_pair_gather",
    )(beta_f, bias_f, y_f)


def _combine_body(x_ref, g_ref, o_ref):
    xv = x_ref[...]
    bg = g_ref[0]
    bb = g_ref[1]
    t = jnp.log1p(xv) * bg + bb
    o_ref[...] = jax.nn.sigmoid(t) - jax.nn.sigmoid(bb)


@functools.partial(jax.jit, static_argnames=("bm",))
def _tc_combine(x2, g3, *, bm):
    m = x2.shape[0]
    return pl.pallas_call(
        _combine_body,
        grid=(m // bm,),
        in_specs=[
            pl.BlockSpec((bm, 128), lambda i: (i, 0)),
            pl.BlockSpec((2, bm, 128), lambda i: (0, i, 0)),
        ],
        out_specs=pl.BlockSpec((bm, 128), lambda i: (i, 0)),
        out_shape=jax.ShapeDtypeStruct((m, 128), jnp.float32),
        name="tc_logsigm_combine",
    )(x2, g3)


def kernel(x, y, beta, bias):
    b, c = x.shape
    n = b * c
    n_tab = beta.shape[1]
    yf = y.reshape(n).astype(jnp.int32)
    g = _sc_gather(beta.reshape(-1), bias.reshape(-1), yf, n=n, n_tab=n_tab)
    x2 = x.reshape(n // 128, 128)
    g3 = g.reshape(2, n // 128, 128)
    out = _tc_combine(x2, g3, bm=1024)
    return out.reshape(b, c)


# speedup vs baseline: 246.5861x; 1.3651x over previous
import functools

import jax
import jax.numpy as jnp
from jax import lax
from jax.experimental import pallas as pl
from jax.experimental.pallas import tpu as pltpu
from jax.experimental.pallas import tpu_sc as plsc

_LANES = 16
_NSUB = 16
_CROWS = 40


def _sc_gather_body(rows_per_sub, beta_hbm, bias_hbm, y_hbm, out_hbm,
                    table_v, idx0, idx1, val0, val1,
                    si0, si1, so0, so1):
    cid = lax.axis_index("c")
    sid = lax.axis_index("s")

    @pl.when(cid == 0)
    def _():
        pltpu.sync_copy(beta_hbm, table_v)

    @pl.when(cid != 0)
    def _():
        pltpu.sync_copy(bias_hbm, table_v)

    base = sid * rows_per_sub
    nch = rows_per_sub // _CROWS
    idx_b = (idx0, idx1)
    val_b = (val0, val1)
    si_b = (si0, si1)
    so_b = (so0, so1)

    def start_in(ch, b):
        pltpu.async_copy(y_hbm.at[pl.ds(base + ch * _CROWS, _CROWS), :],
                         idx_b[b], si_b[b])

    def wait_in(ch, b):
        pltpu.make_async_copy(y_hbm.at[pl.ds(base + ch * _CROWS, _CROWS), :],
                              idx_b[b], si_b[b]).wait()

    def start_out(ch, b):
        pltpu.async_copy(val_b[b],
                         out_hbm.at[cid, pl.ds(base + ch * _CROWS, _CROWS), :],
                         so_b[b])

    def wait_out(ch, b):
        pltpu.make_async_copy(
            val_b[b],
            out_hbm.at[cid, pl.ds(base + ch * _CROWS, _CROWS), :],
            so_b[b]).wait()

    def gather(b):
        iv_ref = idx_b[b]
        ov_ref = val_b[b]

        def row_body(r, _):
            for c in range(128 // _LANES):
                iv = iv_ref[r, pl.ds(c * _LANES, _LANES)]
                ov_ref[r, pl.ds(c * _LANES, _LANES)] = plsc.load_gather(
                    table_v, [iv])
            return 0

        lax.fori_loop(0, _CROWS, row_body, 0, unroll=2)

    start_in(0, 0)

    def body(h, _):
        c0 = 2 * h
        c1 = c0 + 1
        wait_in(c0, 0)
        start_in(c1, 1)

        @pl.when(h > 0)
        def _():
            wait_out(c0 - 2, 0)

        gather(0)
        start_out(c0, 0)
        wait_in(c1, 1)

        @pl.when(c1 + 1 < nch)
        def _():
            start_in(c1 + 1, 0)

        @pl.when(h > 0)
        def _():
            wait_out(c1 - 2, 1)

        gather(1)
        start_out(c1, 1)
        return 0

    lax.fori_loop(0, nch // 2, body, 0)
    wait_out(nch - 2, 0)
    wait_out(nch - 1, 1)


@functools.partial(jax.jit, static_argnames=("n_tab",))
def _sc_gather(beta_f, bias_f, y2, *, n_tab):
    rows = y2.shape[0]
    rows_per_sub = rows // _NSUB
    body = functools.partial(_sc_gather_body, rows_per_sub)
    return pl.kernel(
        body,
        out_type=jax.ShapeDtypeStruct((2, rows, 128), jnp.float32),
        mesh=plsc.VectorSubcoreMesh(core_axis_name="c", subcore_axis_name="s"),
        scratch_types=[
            pltpu.VMEM((n_tab,), jnp.float32),
            pltpu.VMEM((_CROWS, 128), jnp.int32),
            pltpu.VMEM((_CROWS, 128), jnp.int32),
            pltpu.VMEM((_CROWS, 128), jnp.float32),
            pltpu.VMEM((_CROWS, 128), jnp.float32),
            pltpu.SemaphoreType.DMA,
            pltpu.SemaphoreType.DMA,
            pltpu.SemaphoreType.DMA,
            pltpu.SemaphoreType.DMA,
        ],
        compiler_params=pltpu.CompilerParams(needs_layout_passes=False),
        name="sc_pair_gather",
    )(beta_f, bias_f, y2)


def _combine_body(x_ref, g_ref, o_ref):
    xv = x_ref[...]
    bg = g_ref[0]
    bb = g_ref[1]
    t = jnp.log1p(xv) * bg + bb
    o_ref[...] = jax.nn.sigmoid(t) - jax.nn.sigmoid(bb)


@functools.partial(jax.jit, static_argnames=("bm",))
def _tc_combine(x2, g3, *, bm):
    m = x2.shape[0]
    return pl.pallas_call(
        _combine_body,
        grid=(m // bm,),
        in_specs=[
            pl.BlockSpec((bm, 128), lambda i: (i, 0)),
            pl.BlockSpec((2, bm, 128), lambda i: (0, i, 0)),
        ],
        out_specs=pl.BlockSpec((bm, 128), lambda i: (i, 0)),
        out_shape=jax.ShapeDtypeStruct((m, 128), jnp.float32),
        name="tc_logsigm_combine",
    )(x2, g3)


def kernel(x, y, beta, bias):
    b, c = x.shape
    n = b * c
    n_tab = beta.shape[1]
    y2 = y.astype(jnp.int32).reshape(n // 128, 128)
    g3 = _sc_gather(beta.reshape(-1), bias.reshape(-1), y2, n_tab=n_tab)
    x2 = x.reshape(n // 128, 128)
    out = _tc_combine(x2, g3, bm=1024)
    return out.reshape(b, c)


# speedup vs baseline: 338.0073x; 1.3707x over previous
import functools

import jax
import jax.numpy as jnp
from jax import lax
from jax.experimental import pallas as pl
from jax.experimental.pallas import tpu as pltpu
from jax.experimental.pallas import tpu_sc as plsc

_LANES = 16
_NSUB = 16
_CROWS = 40


def _sc_gather_body(rows_per_sub, beta_hbm, bias_hbm, y_hbm, out_hbm,
                    table_v, idx0, idx1, val0, val1,
                    si0, si1, so0, so1):
    cid = lax.axis_index("c")
    sid = lax.axis_index("s")

    @pl.when(cid == 0)
    def _():
        pltpu.sync_copy(beta_hbm, table_v)

    @pl.when(cid != 0)
    def _():
        pltpu.sync_copy(bias_hbm, table_v)

    base = sid * rows_per_sub
    nch = rows_per_sub // _CROWS
    idx_b = (idx0, idx1)
    val_b = (val0, val1)
    si_b = (si0, si1)
    so_b = (so0, so1)

    def start_in(ch, b):
        pltpu.async_copy(y_hbm.at[pl.ds(base + ch * _CROWS, _CROWS), :],
                         idx_b[b], si_b[b])

    def wait_in(ch, b):
        pltpu.make_async_copy(y_hbm.at[pl.ds(base + ch * _CROWS, _CROWS), :],
                              idx_b[b], si_b[b]).wait()

    def start_out(ch, b):
        pltpu.async_copy(val_b[b],
                         out_hbm.at[cid, pl.ds(base + ch * _CROWS, _CROWS), :],
                         so_b[b])

    def wait_out(ch, b):
        pltpu.make_async_copy(
            val_b[b],
            out_hbm.at[cid, pl.ds(base + ch * _CROWS, _CROWS), :],
            so_b[b]).wait()

    def gather(b):
        iv_ref = idx_b[b]
        ov_ref = val_b[b]

        def row_body(r, _):
            ivs = [iv_ref[r, pl.ds(c * _LANES, _LANES)]
                   for c in range(128 // _LANES)]
            gs = [plsc.load_gather(table_v, [iv]) for iv in ivs]
            for c in range(128 // _LANES):
                ov_ref[r, pl.ds(c * _LANES, _LANES)] = gs[c]
            return 0

        lax.fori_loop(0, _CROWS, row_body, 0, unroll=2)

    start_in(0, 0)

    def body(h, _):
        c0 = 2 * h
        c1 = c0 + 1
        wait_in(c0, 0)
        start_in(c1, 1)

        @pl.when(h > 0)
        def _():
            wait_out(c0 - 2, 0)

        gather(0)
        start_out(c0, 0)
        wait_in(c1, 1)

        @pl.when(c1 + 1 < nch)
        def _():
            start_in(c1 + 1, 0)

        @pl.when(h > 0)
        def _():
            wait_out(c1 - 2, 1)

        gather(1)
        start_out(c1, 1)
        return 0

    lax.fori_loop(0, nch // 2, body, 0)
    wait_out(nch - 2, 0)
    wait_out(nch - 1, 1)


@functools.partial(jax.jit, static_argnames=("n_tab",))
def _sc_gather(beta_f, bias_f, y2, *, n_tab):
    rows = y2.shape[0]
    rows_per_sub = rows // _NSUB
    body = functools.partial(_sc_gather_body, rows_per_sub)
    return pl.kernel(
        body,
        out_type=jax.ShapeDtypeStruct((2, rows, 128), jnp.float32),
        mesh=plsc.VectorSubcoreMesh(core_axis_name="c", subcore_axis_name="s"),
        scratch_types=[
            pltpu.VMEM((n_tab,), jnp.float32),
            pltpu.VMEM((_CROWS, 128), jnp.int32),
            pltpu.VMEM((_CROWS, 128), jnp.int32),
            pltpu.VMEM((_CROWS, 128), jnp.float32),
            pltpu.VMEM((_CROWS, 128), jnp.float32),
            pltpu.SemaphoreType.DMA,
            pltpu.SemaphoreType.DMA,
            pltpu.SemaphoreType.DMA,
            pltpu.SemaphoreType.DMA,
        ],
        compiler_params=pltpu.CompilerParams(needs_layout_passes=False),
        name="sc_pair_gather",
    )(beta_f, bias_f, y2)


def _combine_body(x_ref, g_ref, o_ref):
    xv = x_ref[...]
    bg = g_ref[0]
    bb = g_ref[1]
    t = jnp.log1p(xv) * bg + bb
    o_ref[...] = jax.nn.sigmoid(t) - jax.nn.sigmoid(bb)


@functools.partial(jax.jit, static_argnames=("bm",))
def _tc_combine(x2, g3, *, bm):
    m = x2.shape[0]
    return pl.pallas_call(
        _combine_body,
        grid=(m // bm,),
        in_specs=[
            pl.BlockSpec((bm, 128), lambda i: (i, 0)),
            pl.BlockSpec((2, bm, 128), lambda i: (0, i, 0)),
        ],
        out_specs=pl.BlockSpec((bm, 128), lambda i: (i, 0)),
        out_shape=jax.ShapeDtypeStruct((m, 128), jnp.float32),
        name="tc_logsigm_combine",
    )(x2, g3)


def kernel(x, y, beta, bias):
    b, c = x.shape
    n = b * c
    n_tab = beta.shape[1]
    y2 = y.astype(jnp.int32).reshape(n // 128, 128)
    g3 = _sc_gather(beta.reshape(-1), bias.reshape(-1), y2, n_tab=n_tab)
    x2 = x.reshape(n // 128, 128)
    out = _tc_combine(x2, g3, bm=1024)
    return out.reshape(b, c)


# speedup vs baseline: 338.3642x; 1.0011x over previous
import functools

import jax
import jax.numpy as jnp
from jax import lax
from jax.experimental import pallas as pl
from jax.experimental.pallas import tpu as pltpu
from jax.experimental.pallas import tpu_sc as plsc

_LANES = 16
_NSUB = 16
_CROWS = 40


def _sc_gather_body(rows_per_sub, beta_hbm, bias_hbm, y_hbm, out_hbm,
                    table_v, idx0, idx1, val0, val1,
                    si0, si1, so0, so1):
    cid = lax.axis_index("c")
    sid = lax.axis_index("s")

    @pl.when(cid == 0)
    def _():
        pltpu.sync_copy(beta_hbm, table_v)

    @pl.when(cid != 0)
    def _():
        pltpu.sync_copy(bias_hbm, table_v)

    base = sid * rows_per_sub
    nch = rows_per_sub // _CROWS
    idx_b = (idx0, idx1)
    val_b = (val0, val1)
    si_b = (si0, si1)
    so_b = (so0, so1)

    def start_in(ch, b):
        pltpu.async_copy(y_hbm.at[pl.ds(base + ch * _CROWS, _CROWS), :],
                         idx_b[b], si_b[b])

    def wait_in(ch, b):
        pltpu.make_async_copy(y_hbm.at[pl.ds(base + ch * _CROWS, _CROWS), :],
                              idx_b[b], si_b[b]).wait()

    def start_out(ch, b):
        pltpu.async_copy(val_b[b],
                         out_hbm.at[cid, pl.ds(base + ch * _CROWS, _CROWS), :],
                         so_b[b])

    def wait_out(ch, b):
        pltpu.make_async_copy(
            val_b[b],
            out_hbm.at[cid, pl.ds(base + ch * _CROWS, _CROWS), :],
            so_b[b]).wait()

    def gather(b):
        iv_ref = idx_b[b]
        ov_ref = val_b[b]

        @plsc.parallel_loop(0, _CROWS, step=1, unroll=2)
        def _(r):
            ivs = [iv_ref[r, pl.ds(c * _LANES, _LANES)]
                   for c in range(128 // _LANES)]
            gs = [plsc.load_gather(table_v, [iv]) for iv in ivs]
            for c in range(128 // _LANES):
                ov_ref[r, pl.ds(c * _LANES, _LANES)] = gs[c]

    start_in(0, 0)

    def body(h, _):
        c0 = 2 * h
        c1 = c0 + 1
        wait_in(c0, 0)
        start_in(c1, 1)

        @pl.when(h > 0)
        def _():
            wait_out(c0 - 2, 0)

        gather(0)
        start_out(c0, 0)
        wait_in(c1, 1)

        @pl.when(c1 + 1 < nch)
        def _():
            start_in(c1 + 1, 0)

        @pl.when(h > 0)
        def _():
            wait_out(c1 - 2, 1)

        gather(1)
        start_out(c1, 1)
        return 0

    lax.fori_loop(0, nch // 2, body, 0)
    wait_out(nch - 2, 0)
    wait_out(nch - 1, 1)


@functools.partial(jax.jit, static_argnames=("n_tab",))
def _sc_gather(beta_f, bias_f, y2, *, n_tab):
    rows = y2.shape[0]
    rows_per_sub = rows // _NSUB
    body = functools.partial(_sc_gather_body, rows_per_sub)
    return pl.kernel(
        body,
        out_type=jax.ShapeDtypeStruct((2, rows, 128), jnp.float32),
        mesh=plsc.VectorSubcoreMesh(core_axis_name="c", subcore_axis_name="s"),
        scratch_types=[
            pltpu.VMEM((n_tab,), jnp.float32),
            pltpu.VMEM((_CROWS, 128), jnp.int32),
            pltpu.VMEM((_CROWS, 128), jnp.int32),
            pltpu.VMEM((_CROWS, 128), jnp.float32),
            pltpu.VMEM((_CROWS, 128), jnp.float32),
            pltpu.SemaphoreType.DMA,
            pltpu.SemaphoreType.DMA,
            pltpu.SemaphoreType.DMA,
            pltpu.SemaphoreType.DMA,
        ],
        compiler_params=pltpu.CompilerParams(needs_layout_passes=False),
        name="sc_pair_gather",
    )(beta_f, bias_f, y2)


def _combine_body(x_ref, g_ref, o_ref):
    xv = x_ref[...]
    bg = g_ref[0]
    bb = g_ref[1]
    t = jnp.log1p(xv) * bg + bb
    o_ref[...] = jax.nn.sigmoid(t) - jax.nn.sigmoid(bb)


@functools.partial(jax.jit, static_argnames=("bm",))
def _tc_combine(x2, g3, *, bm):
    m = x2.shape[0]
    return pl.pallas_call(
        _combine_body,
        grid=(m // bm,),
        in_specs=[
            pl.BlockSpec((bm, 128), lambda i: (i, 0)),
            pl.BlockSpec((2, bm, 128), lambda i: (0, i, 0)),
        ],
        out_specs=pl.BlockSpec((bm, 128), lambda i: (i, 0)),
        out_shape=jax.ShapeDtypeStruct((m, 128), jnp.float32),
        name="tc_logsigm_combine",
    )(x2, g3)


def kernel(x, y, beta, bias):
    b, c = x.shape
    n = b * c
    n_tab = beta.shape[1]
    y2 = y.astype(jnp.int32).reshape(n // 128, 128)
    g3 = _sc_gather(beta.reshape(-1), bias.reshape(-1), y2, n_tab=n_tab)
    x2 = x.reshape(n // 128, 128)
    out = _tc_combine(x2, g3, bm=1024)
    return out.reshape(b, c)
